# baseline (speedup 1.0000x reference)
import functools

import jax
import jax.numpy as jnp
from jax import lax
from jax.experimental import pallas as pl
from jax.experimental.pallas import tpu as pltpu
from jax.experimental.pallas import tpu_sc as plsc

B = 1024
V = 100
CARD = 1000
D = 128
HID = 128
N = 50
E = 500

_NC = 2
_NS = 16
_NW = _NC * _NS
_NSPLIT = 2
_BS = B // _NSPLIT
_ROWS = _BS * V
_RPW = _ROWS // _NW
_CH = 80
_NCHUNK = _RPW // _CH

_EPAD = 640
_BT = 128
_NP = 64


def _sc_gather_body(table_hbm, idx_hbm, out_hbm, idx_v, buf0, buf1, sem0, sem1):
    wid = lax.axis_index("s") * _NC + lax.axis_index("c")
    base = wid * _RPW
    pltpu.sync_copy(idx_hbm.at[wid], idx_v)

    def start(k, buf, sem):
        pltpu.async_copy(table_hbm.at[idx_v.at[k]], buf, sem)

    def drain_and_store(k, buf, sem):
        pltpu.make_async_copy(table_hbm.at[idx_v.at[k]], buf, sem).wait()
        pltpu.sync_copy(buf, out_hbm.at[pl.ds(base + k * _CH, _CH)])

    start(0, buf0, sem0)

    def body(j, carry):
        k0 = 2 * j
        start(k0 + 1, buf1, sem1)
        drain_and_store(k0, buf0, sem0)

        @pl.when(j < _NCHUNK // 2 - 1)
        def _():
            start(k0 + 2, buf0, sem0)

        drain_and_store(k0 + 1, buf1, sem1)
        return carry

    lax.fori_loop(0, _NCHUNK // 2, body, 0)


@functools.cache
def _sc_gather():
    return functools.partial(
        pl.kernel,
        out_type=jax.ShapeDtypeStruct((_ROWS, D), jnp.float32),
        mesh=plsc.VectorSubcoreMesh(core_axis_name="c", subcore_axis_name="s"),
        scratch_types=[
            pltpu.VMEM((_NCHUNK, _CH), jnp.int32),
            pltpu.VMEM((_CH, D), jnp.float32),
            pltpu.VMEM((_CH, D), jnp.float32),
            pltpu.SemaphoreType.DMA,
            pltpu.SemaphoreType.DMA,
        ],
    )(_sc_gather_body)


def _tc_body(src_ref, dst_ref, att_ref,
             W_z, W_r, W_h, Wl_z, Wl_r, Wl_h,
             bz_ref, br_ref, bh_ref, blz_ref, blr_ref, blh_ref,
             Wc1, bc1, Wc2, bc2, xg_ref, out_ref):
    f32 = jnp.float32

    nodes = lax.broadcasted_iota(jnp.int32, (_NP, _EPAD), 0)
    one_src = (nodes == src_ref[...]).astype(f32)
    one_dst = (nodes == dst_ref[...]).astype(f32)
    deg = jnp.sum(one_dst, axis=1, keepdims=True)
    dinv = jnp.where(deg > 0, lax.rsqrt(deg), 0.0)
    dinv_src = jnp.sum(one_src * dinv, axis=0, keepdims=True)
    dinv_dst = jnp.sum(one_dst * dinv, axis=0, keepdims=True)
    norm = dinv_src * dinv_dst
    A = jnp.dot(one_dst * norm, one_src.T,
                preferred_element_type=f32)[:N, :N]

    Wz_t, Wz_b = Wl_z[:HID], Wl_z[HID:]
    Wr_t, Wr_b = Wl_r[:HID], Wl_r[HID:]
    Wh_t, Wh_b = Wl_h[:HID], Wl_h[HID:]
    Wz_e = jnp.dot(W_z[...], Wz_t, preferred_element_type=f32)
    Wr_e = jnp.dot(W_r[...], Wr_t, preferred_element_type=f32)
    Wh_e = jnp.dot(W_h[...], Wh_t, preferred_element_type=f32)
    bz_e = jnp.dot(bz_ref[...], Wz_t, preferred_element_type=f32) + blz_ref[...]
    br_e = jnp.dot(br_ref[...], Wr_t, preferred_element_type=f32) + blr_ref[...]
    bh_e = jnp.dot(bh_ref[...], Wh_t, preferred_element_type=f32) + blh_ref[...]

    X0 = xg_ref[0]
    X1 = xg_ref[1]

    W0cat = jnp.concatenate([Wz_e, Wh_e], axis=1)
    AX0 = jnp.dot(A, X0, preferred_element_type=f32).reshape(N * _BT, D)
    C0 = jnp.dot(AX0, W0cat, preferred_element_type=f32)
    Z0 = jax.nn.sigmoid(C0[:, :HID] + bz_e)
    T0 = jnp.tanh(C0[:, HID:] + bh_e)
    H1 = (1.0 - Z0) * T0

    W1cat = jnp.concatenate([Wz_e, Wr_e, Wh_e], axis=1)
    Wbcat = jnp.concatenate([Wz_b, Wr_b], axis=1)
    AX1 = jnp.dot(A, X1, preferred_element_type=f32).reshape(N * _BT, D)
    C1 = jnp.dot(AX1, W1cat, preferred_element_type=f32)
    D1 = jnp.dot(H1, Wbcat, preferred_element_type=f32)
    Z1 = jax.nn.sigmoid(C1[:, :HID] + D1[:, :HID] + bz_e)
    R1 = jax.nn.sigmoid(C1[:, HID:2 * HID] + D1[:, HID:] + br_e)
    T1 = jnp.tanh(C1[:, 2 * HID:]
                  + jnp.dot(R1 * H1, Wh_b, preferred_element_type=f32) + bh_e)
    H2 = Z1 * H1 + (1.0 - Z1) * T1

    e = jnp.exp(att_ref[...])
    p = e / jnp.sum(e)
    Hacc = p[0:1, 0:1] * H1 + p[0:1, 1:2] * H2
    pooled = jnp.sum(Hacc.reshape(N, _BT, HID), axis=0) * (1.0 / N)
    hid = jax.nn.relu(jnp.dot(pooled, Wc1[...], preferred_element_type=f32)
                      + bc1[...])
    out_ref[...] = jnp.dot(hid, Wc2[...], preferred_element_type=f32) + bc2[...]


def _full_spec(shape):
    return pl.BlockSpec(shape, lambda i: tuple(0 for _ in shape))


def kernel(x_batch, LOS_batch, template_edge_index, device, emb_table,
           W_z, b_conv_z, W_r, b_conv_r, W_h, b_conv_h,
           Wl_z, bl_z, Wl_r, bl_r, Wl_h, bl_h, att, Wc1, bc1, Wc2, bc2):
    offs = (jnp.arange(V, dtype=jnp.int32) * CARD)[:, None]
    idxT = x_batch.astype(jnp.int32).T + offs

    loops = jnp.arange(N, dtype=jnp.int32)
    pad = jnp.full((_EPAD - E - N,), _NP - 1, jnp.int32)
    src = jnp.concatenate([template_edge_index[0].astype(jnp.int32), loops, pad])
    dst = jnp.concatenate([template_edge_index[1].astype(jnp.int32), loops, pad])

    tc_call = pl.pallas_call(
        _tc_body,
        grid=(_BS // _BT,),
        in_specs=[
            _full_spec((1, _EPAD)),
            _full_spec((1, _EPAD)),
            _full_spec((1, 2)),
            _full_spec((D, HID)), _full_spec((D, HID)), _full_spec((D, HID)),
            _full_spec((2 * HID, HID)), _full_spec((2 * HID, HID)),
            _full_spec((2 * HID, HID)),
            _full_spec((1, HID)), _full_spec((1, HID)), _full_spec((1, HID)),
            _full_spec((1, HID)), _full_spec((1, HID)), _full_spec((1, HID)),
            _full_spec((HID, 2 * HID)), _full_spec((1, 2 * HID)),
            _full_spec((2 * HID, 1)), _full_spec((1, 1)),
            pl.BlockSpec((2, N, _BT * D), lambda i: (0, 0, i)),
        ],
        out_specs=pl.BlockSpec((_BT, 1), lambda i: (i, 0)),
        out_shape=jax.ShapeDtypeStruct((_BS, 1), jnp.float32),
        compiler_params=pltpu.CompilerParams(
            dimension_semantics=("arbitrary",)),
    )

    sc = _sc_gather()
    parts = []
    for s in range(_NSPLIT):
        idx_s = idxT[:, s * _BS:(s + 1) * _BS].reshape(_NW, _NCHUNK, _CH)
        xg = sc(emb_table, idx_s).reshape(2, N, _BS * D)
        parts.append(tc_call(
            src[None, :], dst[None, :], att[None, :].astype(jnp.float32),
            W_z, W_r, W_h, Wl_z, Wl_r, Wl_h,
            b_conv_z[None, :], b_conv_r[None, :], b_conv_h[None, :],
            bl_z[None, :], bl_r[None, :], bl_h[None, :],
            Wc1, bc1[None, :], Wc2, bc2[None, :], xg))
    return jnp.concatenate(parts, axis=0)


# speedup vs baseline: 1.5033x; 1.5033x over previous
import functools

import jax
import jax.numpy as jnp
from jax import lax
from jax.experimental import pallas as pl
from jax.experimental.pallas import tpu as pltpu
from jax.experimental.pallas import tpu_sc as plsc

B = 1024
V = 100
CARD = 1000
D = 128
HID = 128
N = 50
E = 500

_NC = 2
_NS = 16
_NW = _NC * _NS
_NSPLIT = 2
_BS = B // _NSPLIT
_ROWS = _BS * V
_RPW = _ROWS // _NW
_CH = 80
_NCHUNK = _RPW // _CH

_EPAD = 640
_BT = 128
_NP = 64


def _sc_gather_body(table_hbm, idx_hbm, out_hbm, idx_v, buf0, buf1, sem0, sem1):
    wid = lax.axis_index("s") * _NC + lax.axis_index("c")
    base = wid * _RPW
    pltpu.sync_copy(idx_hbm.at[wid], idx_v)

    def start(k, buf, sem):
        pltpu.async_copy(table_hbm.at[idx_v.at[k]], buf, sem)

    def drain_and_store(k, buf, sem):
        pltpu.make_async_copy(table_hbm.at[idx_v.at[k]], buf, sem).wait()
        pltpu.sync_copy(buf, out_hbm.at[pl.ds(base + k * _CH, _CH)])

    start(0, buf0, sem0)

    def body(j, carry):
        k0 = 2 * j
        start(k0 + 1, buf1, sem1)
        drain_and_store(k0, buf0, sem0)

        @pl.when(j < _NCHUNK // 2 - 1)
        def _():
            start(k0 + 2, buf0, sem0)

        drain_and_store(k0 + 1, buf1, sem1)
        return carry

    lax.fori_loop(0, _NCHUNK // 2, body, 0)


@functools.cache
def _sc_gather():
    return functools.partial(
        pl.kernel,
        out_type=jax.ShapeDtypeStruct((_ROWS, D), jnp.float32),
        mesh=plsc.VectorSubcoreMesh(core_axis_name="c", subcore_axis_name="s"),
        scratch_types=[
            pltpu.VMEM((_NCHUNK, _CH), jnp.int32),
            pltpu.VMEM((_CH, D), jnp.float32),
            pltpu.VMEM((_CH, D), jnp.float32),
            pltpu.SemaphoreType.DMA,
            pltpu.SemaphoreType.DMA,
        ],
    )(_sc_gather_body)


def _tc_body(src_ref, dst_ref, att_ref,
             W_z, W_r, W_h, Wl_z, Wl_r, Wl_h,
             bz_ref, br_ref, bh_ref, blz_ref, blr_ref, blh_ref,
             Wc1, bc1, Wc2, bc2, xg_ref, out_ref):
    f32 = jnp.float32

    nodes = lax.broadcasted_iota(jnp.int32, (_NP, _EPAD), 0)
    one_src = (nodes == src_ref[...]).astype(f32)
    one_dst = (nodes == dst_ref[...]).astype(f32)
    deg = jnp.sum(one_dst, axis=1, keepdims=True)
    dinv = jnp.where(deg > 0, lax.rsqrt(deg), 0.0)
    dinv_src = jnp.sum(one_src * dinv, axis=0, keepdims=True)
    dinv_dst = jnp.sum(one_dst * dinv, axis=0, keepdims=True)
    norm = dinv_src * dinv_dst
    A = jnp.dot(one_dst * norm, one_src.T,
                preferred_element_type=f32)[:N, :N]

    Wz_t, Wz_b = Wl_z[:HID], Wl_z[HID:]
    Wr_t, Wr_b = Wl_r[:HID], Wl_r[HID:]
    Wh_t, Wh_b = Wl_h[:HID], Wl_h[HID:]
    Wz_e = jnp.dot(W_z[...], Wz_t, preferred_element_type=f32)
    Wr_e = jnp.dot(W_r[...], Wr_t, preferred_element_type=f32)
    Wh_e = jnp.dot(W_h[...], Wh_t, preferred_element_type=f32)
    bz_e = jnp.dot(bz_ref[...], Wz_t, preferred_element_type=f32) + blz_ref[...]
    br_e = jnp.dot(br_ref[...], Wr_t, preferred_element_type=f32) + blr_ref[...]
    bh_e = jnp.dot(bh_ref[...], Wh_t, preferred_element_type=f32) + blh_ref[...]

    X = xg_ref[...]
    X0 = X[:N].reshape(N, _BT * D)
    X1 = X[N:].reshape(N, _BT * D)

    W0cat = jnp.concatenate([Wz_e, Wh_e], axis=1)
    AX0 = jnp.dot(A, X0, preferred_element_type=f32).reshape(N * _BT, D)
    C0 = jnp.dot(AX0, W0cat, preferred_element_type=f32)
    Z0 = jax.nn.sigmoid(C0[:, :HID] + bz_e)
    T0 = jnp.tanh(C0[:, HID:] + bh_e)
    H1 = (1.0 - Z0) * T0

    W1cat = jnp.concatenate([Wz_e, Wr_e, Wh_e], axis=1)
    Wbcat = jnp.concatenate([Wz_b, Wr_b], axis=1)
    AX1 = jnp.dot(A, X1, preferred_element_type=f32).reshape(N * _BT, D)
    C1 = jnp.dot(AX1, W1cat, preferred_element_type=f32)
    D1 = jnp.dot(H1, Wbcat, preferred_element_type=f32)
    Z1 = jax.nn.sigmoid(C1[:, :HID] + D1[:, :HID] + bz_e)
    R1 = jax.nn.sigmoid(C1[:, HID:2 * HID] + D1[:, HID:] + br_e)
    T1 = jnp.tanh(C1[:, 2 * HID:]
                  + jnp.dot(R1 * H1, Wh_b, preferred_element_type=f32) + bh_e)
    H2 = Z1 * H1 + (1.0 - Z1) * T1

    e = jnp.exp(att_ref[...])
    p = e / jnp.sum(e)
    Hacc = p[0:1, 0:1] * H1 + p[0:1, 1:2] * H2
    pooled = jnp.sum(Hacc.reshape(N, _BT, HID), axis=0) * (1.0 / N)
    hid = jax.nn.relu(jnp.dot(pooled, Wc1[...], preferred_element_type=f32)
                      + bc1[...])
    out_ref[...] = jnp.dot(hid, Wc2[...], preferred_element_type=f32) + bc2[...]


def _full_spec(shape):
    return pl.BlockSpec(shape, lambda i: tuple(0 for _ in shape))


def kernel(x_batch, LOS_batch, template_edge_index, device, emb_table,
           W_z, b_conv_z, W_r, b_conv_r, W_h, b_conv_h,
           Wl_z, bl_z, Wl_r, bl_r, Wl_h, bl_h, att, Wc1, bc1, Wc2, bc2):
    offs = (jnp.arange(V, dtype=jnp.int32) * CARD)[:, None]
    idxT = x_batch.astype(jnp.int32).T + offs

    loops = jnp.arange(N, dtype=jnp.int32)
    pad = jnp.full((_EPAD - E - N,), _NP - 1, jnp.int32)
    src = jnp.concatenate([template_edge_index[0].astype(jnp.int32), loops, pad])
    dst = jnp.concatenate([template_edge_index[1].astype(jnp.int32), loops, pad])

    tc_call = pl.pallas_call(
        _tc_body,
        grid=(_BS // _BT,),
        in_specs=[
            _full_spec((1, _EPAD)),
            _full_spec((1, _EPAD)),
            _full_spec((1, 2)),
            _full_spec((D, HID)), _full_spec((D, HID)), _full_spec((D, HID)),
            _full_spec((2 * HID, HID)), _full_spec((2 * HID, HID)),
            _full_spec((2 * HID, HID)),
            _full_spec((1, HID)), _full_spec((1, HID)), _full_spec((1, HID)),
            _full_spec((1, HID)), _full_spec((1, HID)), _full_spec((1, HID)),
            _full_spec((HID, 2 * HID)), _full_spec((1, 2 * HID)),
            _full_spec((2 * HID, 1)), _full_spec((1, 1)),
            pl.BlockSpec((2 * N, _BT, D), lambda i: (0, i, 0)),
        ],
        out_specs=pl.BlockSpec((_BT, 1), lambda i: (i, 0)),
        out_shape=jax.ShapeDtypeStruct((_BS, 1), jnp.float32),
        compiler_params=pltpu.CompilerParams(
            dimension_semantics=("arbitrary",)),
    )

    sc = _sc_gather()
    parts = []
    for s in range(_NSPLIT):
        idx_s = idxT[:, s * _BS:(s + 1) * _BS].reshape(_NW, _NCHUNK, _CH)
        xg = sc(emb_table, idx_s).reshape(2 * N, _BS, D)
        parts.append(tc_call(
            src[None, :], dst[None, :], att[None, :].astype(jnp.float32),
            W_z, W_r, W_h, Wl_z, Wl_r, Wl_h,
            b_conv_z[None, :], b_conv_r[None, :], b_conv_h[None, :],
            bl_z[None, :], bl_r[None, :], bl_h[None, :],
            Wc1, bc1[None, :], Wc2, bc2[None, :], xg))
    return jnp.concatenate(parts, axis=0)


# speedup vs baseline: 1.5890x; 1.0570x over previous
import functools

import jax
import jax.numpy as jnp
from jax import lax
from jax.experimental import pallas as pl
from jax.experimental.pallas import tpu as pltpu
from jax.experimental.pallas import tpu_sc as plsc

B = 1024
V = 100
CARD = 1000
D = 128
HID = 128
N = 50
E = 500

_NC = 2
_NS = 16
_NW = _NC * _NS
_NSPLIT = 2
_BS = B // _NSPLIT
_ROWS = _BS * V
_RPW = _ROWS // _NW
_CH = 80
_NCHUNK = _RPW // _CH

_EPAD = 640
_BT = 128
_NP = 64


def _sc_gather_body(table_hbm, idx_hbm, out_hbm, idx_v, buf0, buf1, sem0, sem1):
    wid = lax.axis_index("s") * _NC + lax.axis_index("c")
    base = wid * _RPW
    pltpu.sync_copy(idx_hbm.at[wid], idx_v)

    def start(k, buf, sem):
        pltpu.async_copy(table_hbm.at[idx_v.at[k]], buf, sem)

    def drain_and_store(k, buf, sem):
        pltpu.make_async_copy(table_hbm.at[idx_v.at[k]], buf, sem).wait()
        pltpu.sync_copy(buf, out_hbm.at[pl.ds(base + k * _CH, _CH)])

    start(0, buf0, sem0)

    def body(j, carry):
        k0 = 2 * j
        start(k0 + 1, buf1, sem1)
        drain_and_store(k0, buf0, sem0)

        @pl.when(j < _NCHUNK // 2 - 1)
        def _():
            start(k0 + 2, buf0, sem0)

        drain_and_store(k0 + 1, buf1, sem1)
        return carry

    lax.fori_loop(0, _NCHUNK // 2, body, 0)


@functools.cache
def _sc_gather():
    return functools.partial(
        pl.kernel,
        out_type=jax.ShapeDtypeStruct((_ROWS, D), jnp.float32),
        mesh=plsc.VectorSubcoreMesh(core_axis_name="c", subcore_axis_name="s"),
        scratch_types=[
            pltpu.VMEM((_NCHUNK, _CH), jnp.int32),
            pltpu.VMEM((_CH, D), jnp.float32),
            pltpu.VMEM((_CH, D), jnp.float32),
            pltpu.SemaphoreType.DMA,
            pltpu.SemaphoreType.DMA,
        ],
    )(_sc_gather_body)


def _tc_body(src_ref, dst_ref, att_ref,
             W_z, W_r, W_h, Wl_z, Wl_r, Wl_h,
             bz_ref, br_ref, bh_ref, blz_ref, blr_ref, blh_ref,
             Wc1, bc1, Wc2, bc2, xg_ref, out_ref):
    f32 = jnp.float32

    nodes = lax.broadcasted_iota(jnp.int32, (_NP, _EPAD), 0)
    one_src = (nodes == src_ref[...]).astype(f32)
    one_dst = (nodes == dst_ref[...]).astype(f32)
    deg = jnp.sum(one_dst, axis=1, keepdims=True)
    dinv = jnp.where(deg > 0, lax.rsqrt(deg), 0.0)
    dinv_src = jnp.sum(one_src * dinv, axis=0, keepdims=True)
    dinv_dst = jnp.sum(one_dst * dinv, axis=0, keepdims=True)
    norm = dinv_src * dinv_dst
    A = jnp.dot(one_dst * norm, one_src.T,
                preferred_element_type=f32)[:N, :N]

    Wz_t, Wz_b = Wl_z[:HID], Wl_z[HID:]
    Wr_t, Wr_b = Wl_r[:HID], Wl_r[HID:]
    Wh_t, Wh_b = Wl_h[:HID], Wl_h[HID:]
    Wz_e = jnp.dot(W_z[...], Wz_t, preferred_element_type=f32)
    Wr_e = jnp.dot(W_r[...], Wr_t, preferred_element_type=f32)
    Wh_e = jnp.dot(W_h[...], Wh_t, preferred_element_type=f32)
    bz_e = jnp.dot(bz_ref[...], Wz_t, preferred_element_type=f32) + blz_ref[...]
    br_e = jnp.dot(br_ref[...], Wr_t, preferred_element_type=f32) + blr_ref[...]
    bh_e = jnp.dot(bh_ref[...], Wh_t, preferred_element_type=f32) + blh_ref[...]

    bf16 = jnp.bfloat16
    Abf = A.astype(bf16)
    X = xg_ref[...]
    X0 = X[:N].astype(bf16).reshape(N, _BT * D)
    X1 = X[N:].astype(bf16).reshape(N, _BT * D)

    W0cat = jnp.concatenate([Wz_e, Wh_e], axis=1).astype(bf16)
    AX0 = jnp.dot(Abf, X0, preferred_element_type=f32
                  ).astype(bf16).reshape(N * _BT, D)
    C0 = jnp.dot(AX0, W0cat, preferred_element_type=f32)
    Z0 = jax.nn.sigmoid(C0[:, :HID] + bz_e)
    T0 = jnp.tanh(C0[:, HID:] + bh_e)
    H1 = (1.0 - Z0) * T0

    W1cat = jnp.concatenate([Wz_e, Wr_e, Wh_e], axis=1).astype(bf16)
    Wbcat = jnp.concatenate([Wz_b, Wr_b], axis=1).astype(bf16)
    AX1 = jnp.dot(Abf, X1, preferred_element_type=f32
                  ).astype(bf16).reshape(N * _BT, D)
    C1 = jnp.dot(AX1, W1cat, preferred_element_type=f32)
    H1bf = H1.astype(bf16)
    D1 = jnp.dot(H1bf, Wbcat, preferred_element_type=f32)
    Z1 = jax.nn.sigmoid(C1[:, :HID] + D1[:, :HID] + bz_e)
    R1 = jax.nn.sigmoid(C1[:, HID:2 * HID] + D1[:, HID:] + br_e)
    T1 = jnp.tanh(C1[:, 2 * HID:]
                  + jnp.dot((R1 * H1).astype(bf16), Wh_b.astype(bf16),
                            preferred_element_type=f32) + bh_e)
    H2 = Z1 * H1 + (1.0 - Z1) * T1

    e = jnp.exp(att_ref[...])
    p = e / jnp.sum(e)
    Hacc = p[0:1, 0:1] * H1 + p[0:1, 1:2] * H2
    pooled = jnp.sum(Hacc.reshape(N, _BT, HID), axis=0) * (1.0 / N)
    hid = jax.nn.relu(jnp.dot(pooled, Wc1[...], preferred_element_type=f32)
                      + bc1[...])
    out_ref[...] = jnp.dot(hid, Wc2[...], preferred_element_type=f32) + bc2[...]


def _full_spec(shape):
    return pl.BlockSpec(shape, lambda i: tuple(0 for _ in shape))


def kernel(x_batch, LOS_batch, template_edge_index, device, emb_table,
           W_z, b_conv_z, W_r, b_conv_r, W_h, b_conv_h,
           Wl_z, bl_z, Wl_r, bl_r, Wl_h, bl_h, att, Wc1, bc1, Wc2, bc2):
    offs = (jnp.arange(V, dtype=jnp.int32) * CARD)[:, None]
    idxT = x_batch.astype(jnp.int32).T + offs

    loops = jnp.arange(N, dtype=jnp.int32)
    pad = jnp.full((_EPAD - E - N,), _NP - 1, jnp.int32)
    src = jnp.concatenate([template_edge_index[0].astype(jnp.int32), loops, pad])
    dst = jnp.concatenate([template_edge_index[1].astype(jnp.int32), loops, pad])

    tc_call = pl.pallas_call(
        _tc_body,
        grid=(_BS // _BT,),
        in_specs=[
            _full_spec((1, _EPAD)),
            _full_spec((1, _EPAD)),
            _full_spec((1, 2)),
            _full_spec((D, HID)), _full_spec((D, HID)), _full_spec((D, HID)),
            _full_spec((2 * HID, HID)), _full_spec((2 * HID, HID)),
            _full_spec((2 * HID, HID)),
            _full_spec((1, HID)), _full_spec((1, HID)), _full_spec((1, HID)),
            _full_spec((1, HID)), _full_spec((1, HID)), _full_spec((1, HID)),
            _full_spec((HID, 2 * HID)), _full_spec((1, 2 * HID)),
            _full_spec((2 * HID, 1)), _full_spec((1, 1)),
            pl.BlockSpec((2 * N, _BT, D), lambda i: (0, i, 0)),
        ],
        out_specs=pl.BlockSpec((_BT, 1), lambda i: (i, 0)),
        out_shape=jax.ShapeDtypeStruct((_BS, 1), jnp.float32),
        compiler_params=pltpu.CompilerParams(
            dimension_semantics=("arbitrary",)),
    )

    sc = _sc_gather()
    parts = []
    for s in range(_NSPLIT):
        idx_s = idxT[:, s * _BS:(s + 1) * _BS].reshape(_NW, _NCHUNK, _CH)
        xg = sc(emb_table, idx_s).reshape(2 * N, _BS, D)
        parts.append(tc_call(
            src[None, :], dst[None, :], att[None, :].astype(jnp.float32),
            W_z, W_r, W_h, Wl_z, Wl_r, Wl_h,
            b_conv_z[None, :], b_conv_r[None, :], b_conv_h[None, :],
            bl_z[None, :], bl_r[None, :], bl_h[None, :],
            Wc1, bc1[None, :], Wc2, bc2[None, :], xg))
    return jnp.concatenate(parts, axis=0)
